# baseline (speedup 1.0000x reference)
import functools

import jax
import jax.numpy as jnp
from jax import lax
from jax.experimental import pallas as pl
from jax.experimental.pallas import tpu as pltpu


BN_EPS = 1e-5
F_IN = 13
H1P = 16
H2P = 8
P_ROWS, P_COLS = 48, 16
G = 8
W8_ROWS, W8_COLS = G * H1P, G * F_IN
W2X_ROWS, W2X_COLS = G * H2P, G * H1P
A_ROWS = W8_ROWS + W2X_ROWS + G
SROWS = 8


def _fused_kernel(x_ref, a_ref, p_ref, o_ref, hc_ref,
                  s1_ref, q1_ref, s2_ref, q2_ref,
                  *, n_valid, tile_n, masked):
    phase = pl.program_id(0)
    i = pl.program_id(1)
    tpr = tile_n // G
    inv_n = jnp.float32(1.0 / n_valid)

    w8 = a_ref[0:W8_ROWS, 0:W8_COLS]
    w2x = a_ref[W8_ROWS:W8_ROWS + W2X_ROWS, 0:W2X_COLS]
    w3x = a_ref[W8_ROWS + W2X_ROWS:A_ROWS, 0:W2X_ROWS]
    g1r = a_ref[200:201, 0:W8_ROWS]
    be1r = a_ref[201:202, 0:W8_ROWS]
    g2r = a_ref[202:203, 0:W2X_ROWS]
    be2r = a_ref[203:204, 0:W2X_ROWS]
    w3r = a_ref[204:205, 0:W2X_ROWS]
    b3 = p_ref[40:41, 4:5]

    ones8 = jnp.ones((SROWS, tpr), jnp.float32)

    if masked:
        r_iota = lax.broadcasted_iota(jnp.int32, (tpr, W8_ROWS), 0)
        p_iota = lax.broadcasted_iota(jnp.int32, (tpr, W8_ROWS), 1)
        nglob1 = G * (i * tpr + r_iota) + p_iota // H1P
        mask1 = (nglob1 < n_valid).astype(jnp.float32)
        r2 = lax.broadcasted_iota(jnp.int32, (tpr, W2X_ROWS), 0)
        p2 = lax.broadcasted_iota(jnp.int32, (tpr, W2X_ROWS), 1)
        nglob2 = G * (i * tpr + r2) + p2 // H2P
        mask2 = (nglob2 < n_valid).astype(jnp.float32)
    else:
        mask1 = mask2 = None

    def scale_shift(srow, qrow, gr, ber, kw):
        sf = srow[:, 0:kw]
        qf = qrow[:, 0:kw]
        for k in range(1, G):
            sf = sf + srow[:, k * kw:(k + 1) * kw]
            qf = qf + qrow[:, k * kw:(k + 1) * kw]
        mean = sf * inv_n
        var = jnp.maximum(qf * inv_n - mean * mean, 0.0)
        rs = lax.rsqrt(var + BN_EPS)
        a = jnp.concatenate([rs] * G, axis=1)
        m = jnp.concatenate([mean] * G, axis=1)
        return gr * a, ber - m * gr * a

    @pl.when(jnp.logical_and(phase == 0, i == 0))
    def _init():
        s1_ref[...] = jnp.zeros_like(s1_ref)
        q1_ref[...] = jnp.zeros_like(q1_ref)
        s2_ref[...] = jnp.zeros_like(s2_ref)
        q2_ref[...] = jnp.zeros_like(q2_ref)

    @pl.when(phase == 0)
    def _phase0():
        x_blk = x_ref[...]
        h1 = lax.dot_general(x_blk, w8, (((1,), (1,)), ((), ())),
                             preferred_element_type=jnp.float32)
        hc_ref[i] = h1.astype(jnp.bfloat16)
        hm = h1 * mask1 if masked else h1
        s1_ref[...] += lax.dot_general(ones8, hm, (((1,), (0,)), ((), ())),
                                       preferred_element_type=jnp.float32)
        q1_ref[...] += lax.dot_general(ones8, hm * h1, (((1,), (0,)), ((), ())),
                                       preferred_element_type=jnp.float32)

    @pl.when(phase == 1)
    def _phase1():
        a1, c1 = scale_shift(s1_ref[0:1], q1_ref[0:1], g1r, be1r, H1P)
        h1 = hc_ref[i].astype(jnp.float32)
        h1a16 = jnp.maximum(h1 * a1 + c1, 0.0).astype(jnp.bfloat16)
        h2 = lax.dot_general(h1a16, w2x.astype(jnp.bfloat16),
                             (((1,), (1,)), ((), ())),
                             preferred_element_type=jnp.float32)
        hc_ref[i, :, 0:W2X_ROWS] = h2.astype(jnp.bfloat16)
        hm2 = h2 * mask2 if masked else h2
        s2_ref[...] += lax.dot_general(ones8, hm2, (((1,), (0,)), ((), ())),
                                       preferred_element_type=jnp.float32)
        q2_ref[...] += lax.dot_general(ones8, hm2 * h2, (((1,), (0,)), ((), ())),
                                       preferred_element_type=jnp.float32)

    @pl.when(phase == 2)
    def _phase2():
        h2 = hc_ref[i, :, 0:W2X_ROWS].astype(jnp.float32)
        a2, c2 = scale_shift(s2_ref[0:1], q2_ref[0:1], g2r, be2r, H2P)
        h2a = jnp.maximum(h2 * a2 + c2, 0.0)
        h3 = lax.dot_general(h2a, w3x, (((1,), (1,)), ((), ())),
                             preferred_element_type=jnp.float32) + b3
        o_ref[...] = jax.nn.sigmoid(h3)


def _round_up(a: int, b: int) -> int:
    return (a + b - 1) // b * b


def _forward(x, packed_params, *, tile_n=65536):
    n, f = x.shape
    assert f == F_IN, f

    if n <= tile_n:
        tile = _round_up(max(n, 1), 1024)
    else:
        tile = _round_up(tile_n, 1024)
    padded_n = _round_up(n, tile)
    if padded_n != n:
        x = jnp.pad(x, ((0, padded_n - n), (0, 0)))
    num_tiles = padded_n // tile
    tpr = tile // G
    last = num_tiles - 1

    xp = x.reshape(padded_n // G, G * F_IN)

    eye8 = jnp.eye(G, dtype=jnp.float32)
    w8 = jnp.kron(eye8, packed_params[0:H1P, 0:F_IN])
    w2x = jnp.kron(eye8, packed_params[16:24, 0:H1P])
    w3x = jnp.kron(eye8, packed_params[40:48, 3:4].T)
    g1r = jnp.tile(packed_params[24:40, 1], G)[None, :]
    be1r = jnp.tile(packed_params[24:40, 2], G)[None, :]
    g2r = jnp.tile(packed_params[40:48, 1], G)[None, :]
    be2r = jnp.tile(packed_params[40:48, 2], G)[None, :]
    w3r = jnp.tile(packed_params[40:48, 3], G)[None, :]
    aug = jnp.zeros((208, W8_ROWS), jnp.float32)
    aug = aug.at[0:W8_ROWS, 0:W8_COLS].set(w8)
    aug = aug.at[W8_ROWS:W8_ROWS + W2X_ROWS, 0:W2X_COLS].set(w2x)
    aug = aug.at[W8_ROWS + W2X_ROWS:A_ROWS, 0:W2X_ROWS].set(w3x)
    aug = aug.at[200:201, 0:W8_ROWS].set(g1r)
    aug = aug.at[201:202, 0:W8_ROWS].set(be1r)
    aug = aug.at[202:203, 0:W2X_ROWS].set(g2r)
    aug = aug.at[203:204, 0:W2X_ROWS].set(be2r)
    aug = aug.at[204:205, 0:W2X_ROWS].set(w3r)

    out = pl.pallas_call(
        functools.partial(_fused_kernel, n_valid=n, tile_n=tile,
                          masked=padded_n != n),
        out_shape=jax.ShapeDtypeStruct((padded_n // G, G), jnp.float32),
        grid=(3, num_tiles),
        in_specs=[
            pl.BlockSpec((tpr, G * F_IN),
                         lambda p, i: (jnp.where(p == 0, i, last), 0)),
            pl.BlockSpec((208, W8_ROWS), lambda p, i: (0, 0)),
            pl.BlockSpec((P_ROWS, P_COLS), lambda p, i: (0, 0)),
        ],
        out_specs=pl.BlockSpec((tpr, G),
                               lambda p, i: (jnp.where(p == 2, i, 0), 0)),
        scratch_shapes=[
            pltpu.VMEM((num_tiles, tpr, W8_ROWS), jnp.bfloat16),
            pltpu.VMEM((SROWS, W8_ROWS), jnp.float32),
            pltpu.VMEM((SROWS, W8_ROWS), jnp.float32),
            pltpu.VMEM((SROWS, W2X_ROWS), jnp.float32),
            pltpu.VMEM((SROWS, W2X_ROWS), jnp.float32),
        ],
        compiler_params=pltpu.CompilerParams(
            dimension_semantics=("arbitrary", "arbitrary"),
            vmem_limit_bytes=56 * 1024 * 1024,
        ),
    )(xp, aug, packed_params)

    return out.reshape(padded_n, 1)[:n]


def kernel(x, packed_params):
    return _forward(x, packed_params)


# speedup vs baseline: 9.9188x; 9.9188x over previous
import functools

import jax
import jax.numpy as jnp
from jax import lax
from jax.experimental import pallas as pl
from jax.experimental.pallas import tpu as pltpu


BN_EPS = 1e-5
F_IN = 13
H1P = 16
H2P = 8
P_ROWS, P_COLS = 48, 16


def _fused_kernel(x_ref, p_ref, o_ref, hc_ref, s1_ref, q1_ref, s2_ref, q2_ref,
                  *, n_valid, tile_n, masked):
    phase = pl.program_id(0)
    i = pl.program_id(1)
    inv_n = jnp.float32(1.0 / n_valid)

    w1 = p_ref[0:H1P, 0:F_IN]
    w2 = p_ref[16:24, 0:H1P]
    g1 = p_ref[24:40, 1:2]
    be1 = p_ref[24:40, 2:3]
    g2 = p_ref[40:48, 1:2]
    be2 = p_ref[40:48, 2:3]
    w3c = p_ref[40:48, 3:4]
    b3 = p_ref[40:41, 4:5]

    if masked:
        lane = lax.broadcasted_iota(jnp.int32, (H1P, tile_n), 1)
        mask = ((i * tile_n + lane) < n_valid).astype(jnp.float32)
    else:
        mask = None

    def bn_scale_shift(s, q, gamma, beta):
        mean = s * inv_n
        var = jnp.maximum(q * inv_n - mean * mean, 0.0)
        a = gamma * lax.rsqrt(var + BN_EPS)
        return a, beta - mean * a

    @pl.when(jnp.logical_and(phase == 0, i == 0))
    def _init():
        s1_ref[...] = jnp.zeros_like(s1_ref)
        q1_ref[...] = jnp.zeros_like(q1_ref)
        s2_ref[...] = jnp.zeros_like(s2_ref)
        q2_ref[...] = jnp.zeros_like(q2_ref)

    @pl.when(phase == 0)
    def _phase0():
        x_blk = x_ref[...]
        h1 = lax.dot_general(w1, x_blk, (((1,), (0,)), ((), ())),
                             preferred_element_type=jnp.float32)
        hc_ref[i] = h1.astype(jnp.bfloat16)
        hm = h1 * mask if masked else h1
        s1_ref[...] += jnp.sum(hm, axis=-1, keepdims=True)
        q1_ref[...] += jnp.sum(hm * h1, axis=-1, keepdims=True)

    @pl.when(phase == 1)
    def _phase1():
        a1, c1 = bn_scale_shift(s1_ref[...], q1_ref[...], g1, be1)
        h1 = hc_ref[i].astype(jnp.float32)
        h1a16 = jnp.maximum(h1 * a1 + c1, 0.0).astype(jnp.bfloat16)
        h2 = lax.dot_general(w2.astype(jnp.bfloat16), h1a16,
                             (((1,), (0,)), ((), ())),
                             preferred_element_type=jnp.float32)
        hc_ref[i, 0:H2P, :] = h2.astype(jnp.bfloat16)
        hm2 = h2 * mask[0:H2P, :] if masked else h2
        s2_ref[...] += jnp.sum(hm2, axis=-1, keepdims=True)
        q2_ref[...] += jnp.sum(hm2 * h2, axis=-1, keepdims=True)

    @pl.when(phase == 2)
    def _phase2():
        h2 = hc_ref[i, 0:H2P, :].astype(jnp.float32)
        a2, c2 = bn_scale_shift(s2_ref[...], q2_ref[...], g2, be2)
        h2a = jnp.maximum(h2 * a2 + c2, 0.0)
        h3 = lax.dot_general(w3c, h2a, (((0,), (0,)), ((), ())),
                             preferred_element_type=jnp.float32) + b3
        o_ref[...] = jax.nn.sigmoid(h3)


def _round_up(a: int, b: int) -> int:
    return (a + b - 1) // b * b


def _forward(x, packed_params, *, tile_n=131072):
    n, f = x.shape
    assert f == F_IN, f

    if n <= tile_n:
        tile = _round_up(max(n, 1), 8)
    else:
        tile = _round_up(tile_n, 128)
    padded_n = _round_up(n, tile)
    if padded_n != n:
        x = jnp.pad(x, ((0, padded_n - n), (0, 0)))
    num_tiles = padded_n // tile
    last = num_tiles - 1
    xt = x.T

    out = pl.pallas_call(
        functools.partial(_fused_kernel, n_valid=n, tile_n=tile,
                          masked=padded_n != n),
        out_shape=jax.ShapeDtypeStruct((1, padded_n), jnp.float32),
        grid=(3, num_tiles),
        in_specs=[
            pl.BlockSpec((F_IN, tile),
                         lambda p, i: (0, jnp.where(p == 0, i, last))),
            pl.BlockSpec((P_ROWS, P_COLS), lambda p, i: (0, 0)),
        ],
        out_specs=pl.BlockSpec((1, tile),
                               lambda p, i: (0, jnp.where(p == 2, i, 0))),
        scratch_shapes=[
            pltpu.VMEM((num_tiles, H1P, tile), jnp.bfloat16),
            pltpu.VMEM((H1P, 1), jnp.float32),
            pltpu.VMEM((H1P, 1), jnp.float32),
            pltpu.VMEM((H2P, 1), jnp.float32),
            pltpu.VMEM((H2P, 1), jnp.float32),
        ],
        compiler_params=pltpu.CompilerParams(
            dimension_semantics=("arbitrary", "arbitrary"),
            vmem_limit_bytes=56 * 1024 * 1024,
        ),
    )(xt, packed_params)

    return out[:, :n].T


def kernel(x, packed_params):
    return _forward(x, packed_params)
